# Initial kernel scaffold; baseline (speedup 1.0000x reference)
import jax
import jax.numpy as jnp
from jax.experimental import pallas as pl


def kernel(word, table):
    raise NotImplementedError("write your pallas kernel here")


# speedup vs baseline: 1.4767x; 1.4767x over previous
import functools

import jax
import jax.numpy as jnp
from jax import lax
from jax.experimental import pallas as pl
from jax.experimental.pallas import tpu as pltpu
from jax.experimental.pallas import tpu_sc as plsc


def _make_gather(B, V, D, NC, NS):
    NW = NC * NS
    b_per_w = B // NW
    STEP = 128
    K = 8
    steps_per_w = b_per_w // STEP
    G = steps_per_w // K
    GROUP = STEP * K

    mesh = plsc.VectorSubcoreMesh(core_axis_name="c", subcore_axis_name="s")

    @functools.partial(
        pl.kernel,
        mesh=mesh,
        compiler_params=pltpu.CompilerParams(use_tc_tiling_on_sc=False),
        out_type=jax.ShapeDtypeStruct((B, D), jnp.float32),
        scratch_types=[
            pltpu.VMEM((steps_per_w, STEP), jnp.int32),
            pltpu.VMEM((GROUP, D), jnp.float32),
            pltpu.SemaphoreType.DMA,
        ],
    )
    def gather(idx_hbm, table_hbm, out_hbm, idx_v, rows_v, sem):
        wid = lax.axis_index("s") * NC + lax.axis_index("c")
        pltpu.sync_copy(idx_hbm.at[pl.ds(wid * steps_per_w, steps_per_w)], idx_v)
        out_base = wid * b_per_w

        def group_body(g, _):
            copies = []
            for b in range(K):
                cp = pltpu.async_copy(
                    table_hbm.at[idx_v.at[g * K + b]],
                    rows_v.at[pl.ds(b * STEP, STEP)],
                    sem,
                )
                copies.append(cp)
            for cp in copies:
                cp.wait()
            pltpu.sync_copy(rows_v, out_hbm.at[pl.ds(out_base + g * GROUP, GROUP)])
            return 0

        lax.fori_loop(0, G, group_body, 0)

    return gather


def kernel(word, table):
    R, S = word.shape
    V, D = table.shape
    B = R * S
    info = plsc.get_sparse_core_info()
    NC, NS = info.num_cores, info.num_subcores

    STEP = 128
    idx2d = word.reshape(B // STEP, STEP).astype(jnp.int32)
    out = _make_gather(B, V, D, NC, NS)(idx2d, table)
    return out.reshape(R, S, D)


# speedup vs baseline: 1.5003x; 1.0160x over previous
import functools

import jax
import jax.numpy as jnp
from jax import lax
from jax.experimental import pallas as pl
from jax.experimental.pallas import tpu as pltpu
from jax.experimental.pallas import tpu_sc as plsc

_STEP = 128
_K = 8
_NB = 3


def _make_gather(B, V, D, NC, NS):
    NW = NC * NS
    b_per_w = B // NW
    steps_per_w = b_per_w // _STEP
    G = steps_per_w // _K
    GROUP = _STEP * _K

    mesh = plsc.VectorSubcoreMesh(core_axis_name="c", subcore_axis_name="s")

    @functools.partial(
        pl.kernel,
        mesh=mesh,
        compiler_params=pltpu.CompilerParams(use_tc_tiling_on_sc=False),
        out_type=jax.ShapeDtypeStruct((B, D), jnp.float32),
        scratch_types=[
            pltpu.VMEM((steps_per_w, _STEP), jnp.int32),
            pltpu.VMEM((_NB * GROUP, D), jnp.float32),
            pltpu.SemaphoreType.DMA((_NB,)),
            pltpu.SemaphoreType.DMA((_NB,)),
        ],
    )
    def gather(idx_hbm, table_hbm, out_hbm, idx_v, rows_v, gsem, osem):
        wid = lax.axis_index("s") * NC + lax.axis_index("c")
        pltpu.sync_copy(idx_hbm.at[pl.ds(wid * steps_per_w, steps_per_w)], idx_v)
        out_base = wid * b_per_w

        def fire(g, slot):
            for b in range(_K):
                pltpu.async_copy(
                    table_hbm.at[idx_v.at[g * _K + b]],
                    rows_v.at[pl.ds(slot * GROUP + b * _STEP, _STEP)],
                    gsem.at[slot],
                )

        def wait_gathers(g, slot):
            for b in range(_K):
                pltpu.make_async_copy(
                    table_hbm.at[idx_v.at[g * _K + b]],
                    rows_v.at[pl.ds(slot * GROUP + b * _STEP, _STEP)],
                    gsem.at[slot],
                ).wait()

        def write(g, slot):
            pltpu.async_copy(
                rows_v.at[pl.ds(slot * GROUP, GROUP)],
                out_hbm.at[pl.ds(out_base + g * GROUP, GROUP)],
                osem.at[slot],
            )

        def wait_write(g, slot):
            pltpu.make_async_copy(
                rows_v.at[pl.ds(slot * GROUP, GROUP)],
                out_hbm.at[pl.ds(out_base + g * GROUP, GROUP)],
                osem.at[slot],
            ).wait()

        fire(0, 0)
        fire(1, 1)

        def body(g, _):
            slot = lax.rem(g, _NB)
            nslot = lax.rem(g + 2, _NB)

            @pl.when(g + 2 < G)
            def _fire_ahead():
                @pl.when(g >= 1)
                def _drain_prev_write():
                    wait_write(g - 1, nslot)

                fire(g + 2, nslot)

            wait_gathers(g, slot)
            write(g, slot)
            return 0

        lax.fori_loop(0, G, body, 0)

        for g in (G - 3, G - 2, G - 1):
            wait_write(g, lax.rem(jnp.int32(g), _NB))

    return gather


def kernel(word, table):
    R, S = word.shape
    V, D = table.shape
    B = R * S
    info = plsc.get_sparse_core_info()
    NC, NS = info.num_cores, info.num_subcores

    idx2d = word.reshape(B // _STEP, _STEP).astype(jnp.int32)
    out = _make_gather(B, V, D, NC, NS)(idx2d, table)
    return out.reshape(R, S, D)


# speedup vs baseline: 1.5011x; 1.0006x over previous
import functools

import jax
import jax.numpy as jnp
from jax import lax
from jax.experimental import pallas as pl
from jax.experimental.pallas import tpu as pltpu
from jax.experimental.pallas import tpu_sc as plsc

_STEP = 1024
_K = 1
_NB = 3


def _make_gather(B, V, D, NC, NS):
    NW = NC * NS
    b_per_w = B // NW
    steps_per_w = b_per_w // _STEP
    G = steps_per_w // _K
    GROUP = _STEP * _K

    mesh = plsc.VectorSubcoreMesh(core_axis_name="c", subcore_axis_name="s")

    @functools.partial(
        pl.kernel,
        mesh=mesh,
        compiler_params=pltpu.CompilerParams(use_tc_tiling_on_sc=False),
        out_type=jax.ShapeDtypeStruct((B, D), jnp.float32),
        scratch_types=[
            pltpu.VMEM((b_per_w,), jnp.int32),
            pltpu.VMEM((_NB * GROUP, D), jnp.float32),
            pltpu.SemaphoreType.DMA((_NB,)),
            pltpu.SemaphoreType.DMA((_NB,)),
        ],
    )
    def gather(idx_hbm, table_hbm, out_hbm, idx_v, rows_v, gsem, osem):
        wid = lax.axis_index("s") * NC + lax.axis_index("c")
        pltpu.sync_copy(idx_hbm.at[pl.ds(wid * b_per_w, b_per_w)], idx_v)
        out_base = wid * b_per_w

        def fire(g, slot):
            for b in range(_K):
                pltpu.async_copy(
                    table_hbm.at[idx_v.at[pl.ds((g * _K + b) * _STEP, _STEP)]],
                    rows_v.at[pl.ds(slot * GROUP + b * _STEP, _STEP)],
                    gsem.at[slot],
                )

        def wait_gathers(g, slot):
            for b in range(_K):
                pltpu.make_async_copy(
                    table_hbm.at[idx_v.at[pl.ds((g * _K + b) * _STEP, _STEP)]],
                    rows_v.at[pl.ds(slot * GROUP + b * _STEP, _STEP)],
                    gsem.at[slot],
                ).wait()

        def write(g, slot):
            pltpu.async_copy(
                rows_v.at[pl.ds(slot * GROUP, GROUP)],
                out_hbm.at[pl.ds(out_base + g * GROUP, GROUP)],
                osem.at[slot],
            )

        def wait_write(g, slot):
            pltpu.make_async_copy(
                rows_v.at[pl.ds(slot * GROUP, GROUP)],
                out_hbm.at[pl.ds(out_base + g * GROUP, GROUP)],
                osem.at[slot],
            ).wait()

        fire(0, 0)
        fire(1, 1)

        def body(g, _):
            slot = lax.rem(g, _NB)
            nslot = lax.rem(g + 2, _NB)

            @pl.when(g + 2 < G)
            def _fire_ahead():
                @pl.when(g >= 1)
                def _drain_prev_write():
                    wait_write(g - 1, nslot)

                fire(g + 2, nslot)

            wait_gathers(g, slot)
            write(g, slot)
            return 0

        lax.fori_loop(0, G, body, 0)

        for g in (G - 3, G - 2, G - 1):
            wait_write(g, lax.rem(jnp.int32(g), _NB))

    return gather


def kernel(word, table):
    R, S = word.shape
    V, D = table.shape
    B = R * S
    info = plsc.get_sparse_core_info()
    NC, NS = info.num_cores, info.num_subcores

    idx_flat = word.reshape(B).astype(jnp.int32)
    out = _make_gather(B, V, D, NC, NS)(idx_flat, table)
    return out.reshape(R, S, D)
